# Initial kernel scaffold; baseline (speedup 1.0000x reference)
import jax
import jax.numpy as jnp
from jax.experimental import pallas as pl


def kernel(x, edge_index, W1, b1, W2, b2):
    raise NotImplementedError("write your pallas kernel here")


# speedup vs baseline: 19.7066x; 19.7066x over previous
import functools

import jax
import jax.numpy as jnp
from jax import lax
from jax.experimental import pallas as pl
from jax.experimental.pallas import tpu as pltpu
from jax.experimental.pallas import tpu_sc as plsc

N = 10000
NP = 10240
F = 128
H = 16
NC = 2
NS = 16
LANES = 16

CH = 1408
GPT = 8
EW = CH * GPT
EP = NC * NS * EW
RPT = NP // NS
DPT = NP // (NC * NS)
DEGR = EP // (NS * 128)

_mesh = plsc.VectorSubcoreMesh(core_axis_name="c", subcore_axis_name="s")


@functools.partial(
    pl.kernel,
    out_type=jax.ShapeDtypeStruct((NP, H), jnp.float32),
    mesh=_mesh,
    scratch_types=[
        pltpu.VMEM((DEGR, 128), jnp.int32),
        pltpu.VMEM((128,), jnp.float32),
        pltpu.VMEM((RPT,), jnp.float32),
        pltpu.VMEM((DPT,), jnp.float32),
        pltpu.VMEM((DPT, H), jnp.float32),
        pltpu.VMEM_SHARED((NP,), jnp.float32),
    ],
)
def _deg_dis(dst_hbm, out_hbm, dstv, ones, zb, degb, disb, deg_sh):
    c = lax.axis_index("c")
    s = lax.axis_index("s")

    def zb_init(i, _):
        zb[pl.ds(i * LANES, LANES)] = jnp.zeros((LANES,), jnp.float32)
        return 0

    lax.fori_loop(0, RPT // LANES, zb_init, 0)
    for k in range(128 // LANES):
        ones[pl.ds(k * LANES, LANES)] = jnp.full((LANES,), 1.0, jnp.float32)
    pltpu.sync_copy(zb.at[pl.ds(0, RPT)], deg_sh.at[pl.ds(s * RPT, RPT)])
    plsc.subcore_barrier()

    pltpu.sync_copy(dst_hbm.at[pl.ds(s * DEGR, DEGR)], dstv)

    def scat(r, _):
        pltpu.sync_copy(ones, deg_sh.at[dstv.at[r]], add=True)
        return 0

    lax.fori_loop(0, DEGR, scat, 0)
    plsc.subcore_barrier()

    g0 = (c * NS + s) * DPT
    pltpu.sync_copy(deg_sh.at[pl.ds(g0, DPT)], degb)

    def dis_blk(i, _):
        v = degb[pl.ds(i * LANES, LANES)]
        for k in range(LANES):
            disb[i * LANES + k] = jnp.full((LANES,), v[k], jnp.float32)
        return 0

    lax.fori_loop(0, DPT // LANES, dis_blk, 0)
    pltpu.sync_copy(disb, out_hbm.at[pl.ds(g0, DPT)])


@functools.partial(
    pl.kernel,
    out_type=jax.ShapeDtypeStruct((NC, NP, H), jnp.float32),
    mesh=_mesh,
    scratch_types=[
        pltpu.VMEM((GPT, CH), jnp.int32),
        pltpu.VMEM((EW // 128, 128), jnp.int32),
        pltpu.VMEM((CH, H), jnp.float32),
        pltpu.VMEM_SHARED((NP, H), jnp.float32),
        pltpu.SemaphoreType.DMA,
    ],
    compiler_params=pltpu.CompilerParams(use_tc_tiling_on_sc=False),
)
def _agg(h_hbm, src_hbm, dst_hbm, out_hbm, srcv, dstv, rows, acc_sh, sem):
    c = lax.axis_index("c")
    s = lax.axis_index("s")

    def zrow(i, _):
        rows[i] = jnp.zeros((H,), jnp.float32)
        return 0

    lax.fori_loop(0, RPT, zrow, 0)
    pltpu.sync_copy(rows.at[pl.ds(0, RPT)], acc_sh.at[pl.ds(s * RPT, RPT)])
    plsc.subcore_barrier()

    wid = c * NS + s
    pltpu.sync_copy(src_hbm.at[pl.ds(wid * GPT, GPT)], srcv)
    pltpu.sync_copy(dst_hbm.at[pl.ds(wid * (EW // 128), EW // 128)], dstv)

    def chunk(g, _):
        pltpu.async_copy(h_hbm.at[srcv.at[g]], rows, sem).wait()
        for j in range(CH // 128):
            pltpu.sync_copy(rows.at[pl.ds(j * 128, 128)],
                            acc_sh.at[dstv.at[g * (CH // 128) + j]],
                            add=True)
        return 0

    lax.fori_loop(0, GPT, chunk, 0)
    plsc.subcore_barrier()
    pltpu.sync_copy(acc_sh.at[pl.ds(s * RPT, RPT)],
                    out_hbm.at[c, pl.ds(s * RPT, RPT)])


def _lin1_body(x_ref, w_ref, deg_ref, o_ref, dis_ref):
    dis = lax.rsqrt(jnp.maximum(deg_ref[...], 1.0))
    dis_ref[...] = dis
    h = jnp.dot(x_ref[...], w_ref[...], preferred_element_type=jnp.float32)
    o_ref[...] = h * dis


def _lin2_body(p_ref, dis_ref, b1_ref, w2_ref, o_ref):
    acc = p_ref[0] + p_ref[1]
    out1 = jnp.maximum(acc * dis_ref[...] + b1_ref[...], 0.0)
    h2 = jnp.dot(out1, w2_ref[...], preferred_element_type=jnp.float32)
    o_ref[...] = h2 * dis_ref[...]


def _fin_body(p_ref, dis_ref, b2_ref, o_ref):
    o_ref[...] = (p_ref[0] + p_ref[1]) * dis_ref[...] + b2_ref[...]


def kernel(x, edge_index, W1, b1, W2, b2):
    E = edge_index.shape[1]
    src = edge_index[0].astype(jnp.int32)
    dst = edge_index[1].astype(jnp.int32)
    loop_idx = jnp.arange(N, dtype=jnp.int32)
    padv = jnp.full((EP - E - N,), N, dtype=jnp.int32)
    src_e = jnp.concatenate([src, loop_idx, padv]).reshape(EP // CH, CH)
    dst_e = jnp.concatenate([dst, loop_idx, padv]).reshape(EP // 128, 128)

    x_pad = jnp.pad(x, ((0, NP - N), (0, 0)))
    w2p = jnp.pad(W2, ((0, 0), (0, H - W2.shape[1])))
    b1r = b1.reshape(1, H)
    b2r = jnp.pad(b2, (0, H - b2.shape[0])).reshape(1, H)

    deg = _deg_dis(dst_e)

    h1, dis = pl.pallas_call(
        _lin1_body,
        out_shape=(
            jax.ShapeDtypeStruct((NP, H), jnp.float32),
            jax.ShapeDtypeStruct((NP, H), jnp.float32),
        ),
    )(x_pad, W1, deg)

    p1 = _agg(h1, src_e, dst_e)

    h2 = pl.pallas_call(
        _lin2_body,
        out_shape=jax.ShapeDtypeStruct((NP, H), jnp.float32),
    )(p1, dis, b1r, w2p)

    p2 = _agg(h2, src_e, dst_e)

    out = pl.pallas_call(
        _fin_body,
        out_shape=jax.ShapeDtypeStruct((NP, H), jnp.float32),
    )(p2, dis, b2r)

    return out[:N, : b2.shape[0]]


# speedup vs baseline: 20.2867x; 1.0294x over previous
import functools

import jax
import jax.numpy as jnp
from jax import lax
from jax.experimental import pallas as pl
from jax.experimental.pallas import tpu as pltpu
from jax.experimental.pallas import tpu_sc as plsc

N = 10000
NP = 10240
F = 128
H = 16
NC = 2
NS = 16
LANES = 16

CH = 1408
GPT = 8
EW = CH * GPT
EP = NC * NS * EW
RPT = NP // NS
DPT = NP // (NC * NS)
DEGR = EP // (NS * CH)

_mesh = plsc.VectorSubcoreMesh(core_axis_name="c", subcore_axis_name="s")


@functools.partial(
    pl.kernel,
    out_type=jax.ShapeDtypeStruct((NP, H), jnp.float32),
    mesh=_mesh,
    scratch_types=[
        pltpu.VMEM((DEGR, CH), jnp.int32),
        pltpu.VMEM((CH,), jnp.float32),
        pltpu.VMEM((RPT,), jnp.float32),
        pltpu.VMEM((DPT,), jnp.float32),
        pltpu.VMEM((DPT, H), jnp.float32),
        pltpu.VMEM_SHARED((NP,), jnp.float32),
    ],
    compiler_params=pltpu.CompilerParams(use_tc_tiling_on_sc=False),
)
def _deg_dis(dst_hbm, out_hbm, dstv, ones, zb, degb, disb, deg_sh):
    c = lax.axis_index("c")
    s = lax.axis_index("s")

    def zb_init(i, _):
        zb[pl.ds(i * LANES, LANES)] = jnp.zeros((LANES,), jnp.float32)
        return 0

    lax.fori_loop(0, RPT // LANES, zb_init, 0)

    def ones_init(i, _):
        ones[pl.ds(i * LANES, LANES)] = jnp.full((LANES,), 1.0, jnp.float32)
        return 0

    lax.fori_loop(0, CH // LANES, ones_init, 0)
    pltpu.sync_copy(zb.at[pl.ds(0, RPT)], deg_sh.at[pl.ds(s * RPT, RPT)])
    plsc.subcore_barrier()

    pltpu.sync_copy(dst_hbm.at[pl.ds(s * DEGR, DEGR)], dstv)

    def scat(r, _):
        pltpu.sync_copy(ones, deg_sh.at[dstv.at[r]], add=True)
        return 0

    lax.fori_loop(0, DEGR, scat, 0)
    plsc.subcore_barrier()

    g0 = (c * NS + s) * DPT
    pltpu.sync_copy(deg_sh.at[pl.ds(g0, DPT)], degb)

    def dis_blk(i, _):
        v = degb[pl.ds(i * LANES, LANES)]
        for k in range(LANES):
            disb[i * LANES + k] = jnp.full((LANES,), v[k], jnp.float32)
        return 0

    lax.fori_loop(0, DPT // LANES, dis_blk, 0)
    pltpu.sync_copy(disb, out_hbm.at[pl.ds(g0, DPT)])


@functools.partial(
    pl.kernel,
    out_type=jax.ShapeDtypeStruct((NC, NP, H), jnp.float32),
    mesh=_mesh,
    scratch_types=[
        pltpu.VMEM((GPT, CH), jnp.int32),
        pltpu.VMEM((GPT, CH), jnp.int32),
        pltpu.VMEM((CH, H), jnp.float32),
        pltpu.VMEM((CH, H), jnp.float32),
        pltpu.VMEM_SHARED((NP, H), jnp.float32),
        pltpu.SemaphoreType.DMA,
        pltpu.SemaphoreType.DMA,
    ],
    compiler_params=pltpu.CompilerParams(use_tc_tiling_on_sc=False),
)
def _agg(h_hbm, src_hbm, dst_hbm, out_hbm, srcv, dstv, rows0, rows1,
         acc_sh, sem0, sem1):
    c = lax.axis_index("c")
    s = lax.axis_index("s")

    def zrow(i, _):
        rows0[i] = jnp.zeros((H,), jnp.float32)
        return 0

    lax.fori_loop(0, RPT, zrow, 0)
    pltpu.sync_copy(rows0.at[pl.ds(0, RPT)], acc_sh.at[pl.ds(s * RPT, RPT)])
    plsc.subcore_barrier()

    wid = c * NS + s
    pltpu.sync_copy(src_hbm.at[pl.ds(wid * GPT, GPT)], srcv)
    pltpu.sync_copy(dst_hbm.at[pl.ds(wid * GPT, GPT)], dstv)

    bufs = (rows0, rows1)
    sems = (sem0, sem1)
    dsc = pltpu.async_copy(h_hbm.at[srcv.at[0]], bufs[0], sems[0])
    for g in range(GPT):
        dsc.wait()
        if g + 1 < GPT:
            nxt = pltpu.async_copy(
                h_hbm.at[srcv.at[g + 1]], bufs[(g + 1) % 2], sems[(g + 1) % 2])
        pltpu.sync_copy(bufs[g % 2], acc_sh.at[dstv.at[g]], add=True)
        if g + 1 < GPT:
            dsc = nxt
    plsc.subcore_barrier()
    pltpu.sync_copy(acc_sh.at[pl.ds(s * RPT, RPT)],
                    out_hbm.at[c, pl.ds(s * RPT, RPT)])


def _lin1_body(x_ref, w_ref, deg_ref, o_ref, dis_ref):
    dis = lax.rsqrt(jnp.maximum(deg_ref[...], 1.0))
    dis_ref[...] = dis
    h = jnp.dot(x_ref[...], w_ref[...], preferred_element_type=jnp.float32)
    o_ref[...] = h * dis


def _lin2_body(p_ref, dis_ref, b1_ref, w2_ref, o_ref):
    acc = p_ref[0] + p_ref[1]
    out1 = jnp.maximum(acc * dis_ref[...] + b1_ref[...], 0.0)
    h2 = jnp.dot(out1, w2_ref[...], preferred_element_type=jnp.float32)
    o_ref[...] = h2 * dis_ref[...]


def _fin_body(p_ref, dis_ref, b2_ref, o_ref):
    o_ref[...] = (p_ref[0] + p_ref[1]) * dis_ref[...] + b2_ref[...]


def kernel(x, edge_index, W1, b1, W2, b2):
    E = edge_index.shape[1]
    src = edge_index[0].astype(jnp.int32)
    dst = edge_index[1].astype(jnp.int32)
    loop_idx = jnp.arange(N, dtype=jnp.int32)
    padv = jnp.full((EP - E - N,), N, dtype=jnp.int32)
    src_e = jnp.concatenate([src, loop_idx, padv]).reshape(EP // CH, CH)
    dst_e = jnp.concatenate([dst, loop_idx, padv]).reshape(EP // CH, CH)

    x_pad = jnp.pad(x, ((0, NP - N), (0, 0)))
    w2p = jnp.pad(W2, ((0, 0), (0, H - W2.shape[1])))
    b1r = b1.reshape(1, H)
    b2r = jnp.pad(b2, (0, H - b2.shape[0])).reshape(1, H)

    deg = _deg_dis(dst_e)

    h1, dis = pl.pallas_call(
        _lin1_body,
        out_shape=(
            jax.ShapeDtypeStruct((NP, H), jnp.float32),
            jax.ShapeDtypeStruct((NP, H), jnp.float32),
        ),
    )(x_pad, W1, deg)

    p1 = _agg(h1, src_e, dst_e)

    h2 = pl.pallas_call(
        _lin2_body,
        out_shape=jax.ShapeDtypeStruct((NP, H), jnp.float32),
    )(p1, dis, b1r, w2p)

    p2 = _agg(h2, src_e, dst_e)

    out = pl.pallas_call(
        _fin_body,
        out_shape=jax.ShapeDtypeStruct((NP, H), jnp.float32),
    )(p2, dis, b2r)

    return out[:N, : b2.shape[0]]


# speedup vs baseline: 56.4883x; 2.7845x over previous
import functools

import jax
import jax.numpy as jnp
from jax import lax
from jax.experimental import pallas as pl
from jax.experimental.pallas import tpu as pltpu
from jax.experimental.pallas import tpu_sc as plsc

N = 10000
NP = 10240
F = 128
H = 16
NC = 2
NS = 16
LANES = 16

CH = 1408
GPT = 8
EW = CH * GPT
EP = NC * NS * EW
RPT = NP // NS
DPT = NP // (NC * NS)
DEGR = EP // (NS * CH)

_mesh = plsc.VectorSubcoreMesh(core_axis_name="c", subcore_axis_name="s")


@functools.partial(
    pl.kernel,
    out_type=jax.ShapeDtypeStruct((NP, H), jnp.float32),
    mesh=_mesh,
    scratch_types=[
        pltpu.VMEM((DEGR, CH), jnp.int32),
        pltpu.VMEM((CH,), jnp.float32),
        pltpu.VMEM((RPT,), jnp.float32),
        pltpu.VMEM((DPT,), jnp.float32),
        pltpu.VMEM((DPT, H), jnp.float32),
        pltpu.VMEM_SHARED((NP,), jnp.float32),
    ],
    compiler_params=pltpu.CompilerParams(use_tc_tiling_on_sc=False),
)
def _deg_dis(dst_hbm, out_hbm, dstv, ones, zb, degb, disb, deg_sh):
    c = lax.axis_index("c")
    s = lax.axis_index("s")

    def zb_init(i, _):
        zb[pl.ds(i * LANES, LANES)] = jnp.zeros((LANES,), jnp.float32)
        return 0

    lax.fori_loop(0, RPT // LANES, zb_init, 0)

    def ones_init(i, _):
        ones[pl.ds(i * LANES, LANES)] = jnp.full((LANES,), 1.0, jnp.float32)
        return 0

    lax.fori_loop(0, CH // LANES, ones_init, 0)
    pltpu.sync_copy(zb.at[pl.ds(0, RPT)], deg_sh.at[pl.ds(s * RPT, RPT)])
    plsc.subcore_barrier()

    pltpu.sync_copy(dst_hbm.at[pl.ds(s * DEGR, DEGR)], dstv)

    def scat(r, _):
        pltpu.sync_copy(ones, deg_sh.at[dstv.at[r]], add=True)
        return 0

    lax.fori_loop(0, DEGR, scat, 0)
    plsc.subcore_barrier()

    g0 = (c * NS + s) * DPT
    pltpu.sync_copy(deg_sh.at[pl.ds(g0, DPT)], degb)

    def dis_blk(i, _):
        v = degb[pl.ds(i * LANES, LANES)]
        for k in range(LANES):
            disb[i * LANES + k] = jnp.full((LANES,), v[k], jnp.float32)
        return 0

    lax.fori_loop(0, DPT // LANES, dis_blk, 0)
    pltpu.sync_copy(disb, out_hbm.at[pl.ds(g0, DPT)])


@functools.partial(
    pl.kernel,
    out_type=jax.ShapeDtypeStruct((NC, NP, H), jnp.float32),
    mesh=_mesh,
    scratch_types=[
        pltpu.VMEM((GPT, CH), jnp.int32),
        pltpu.VMEM((GPT, CH), jnp.int32),
        pltpu.VMEM((CH, H), jnp.float32),
        pltpu.VMEM((CH, H), jnp.float32),
        pltpu.VMEM_SHARED((NP, H), jnp.float32),
        pltpu.SemaphoreType.DMA,
        pltpu.SemaphoreType.DMA,
    ],
    compiler_params=pltpu.CompilerParams(use_tc_tiling_on_sc=False),
)
def _agg(h_hbm, src_hbm, dst_hbm, out_hbm, srcv, dstv, rows0, rows1,
         acc_sh, sem0, sem1):
    c = lax.axis_index("c")
    s = lax.axis_index("s")

    def zrow(i, _):
        rows0[i] = jnp.zeros((H,), jnp.float32)
        return 0

    lax.fori_loop(0, RPT, zrow, 0)
    pltpu.sync_copy(rows0.at[pl.ds(0, RPT)], acc_sh.at[pl.ds(s * RPT, RPT)])
    plsc.subcore_barrier()

    wid = c * NS + s
    pltpu.sync_copy(src_hbm.at[pl.ds(wid * GPT, GPT)], srcv)
    pltpu.sync_copy(dst_hbm.at[pl.ds(wid * GPT, GPT)], dstv)

    bufs = (rows0, rows1)
    sems = (sem0, sem1)
    dsc = pltpu.async_copy(h_hbm.at[srcv.at[0]], bufs[0], sems[0])
    for g in range(GPT):
        dsc.wait()
        if g + 1 < GPT:
            nxt = pltpu.async_copy(
                h_hbm.at[srcv.at[g + 1]], bufs[(g + 1) % 2], sems[(g + 1) % 2])
        pltpu.sync_copy(bufs[g % 2], acc_sh.at[dstv.at[g]], add=True)
        if g + 1 < GPT:
            dsc = nxt
    plsc.subcore_barrier()
    pltpu.sync_copy(acc_sh.at[pl.ds(s * RPT, RPT)],
                    out_hbm.at[c, pl.ds(s * RPT, RPT)])


def _lin1_body(x_ref, w_ref, deg_ref, o_ref, dis_ref):
    dis = lax.rsqrt(jnp.maximum(deg_ref[...], 1.0))
    dis_ref[...] = dis
    h = jnp.dot(x_ref[...], w_ref[...], preferred_element_type=jnp.float32)
    o_ref[...] = h * dis


def _lin2_body(p_ref, dis_ref, b1_ref, w2_ref, o_ref):
    acc = p_ref[0] + p_ref[1]
    out1 = jnp.maximum(acc * dis_ref[...] + b1_ref[...], 0.0)
    h2 = jnp.dot(out1, w2_ref[...], preferred_element_type=jnp.float32)
    o_ref[...] = h2 * dis_ref[...]


def _fin_body(p_ref, dis_ref, b2_ref, o_ref):
    o_ref[...] = (p_ref[0] + p_ref[1]) * dis_ref[...] + b2_ref[...]


def kernel(x, edge_index, W1, b1, W2, b2):
    E = edge_index.shape[1]
    src = edge_index[0].astype(jnp.int32)
    dst = edge_index[1].astype(jnp.int32)
    loop_idx = jnp.arange(N, dtype=jnp.int32)
    padv = N + jnp.arange(EP - E - N, dtype=jnp.int32) % (NP - N)
    src_e = jnp.concatenate([src, loop_idx, padv]).reshape(EP // CH, CH)
    dst_e = jnp.concatenate([dst, loop_idx, padv]).reshape(EP // CH, CH)

    x_pad = jnp.pad(x, ((0, NP - N), (0, 0)))
    w2p = jnp.pad(W2, ((0, 0), (0, H - W2.shape[1])))
    b1r = b1.reshape(1, H)
    b2r = jnp.pad(b2, (0, H - b2.shape[0])).reshape(1, H)

    deg = _deg_dis(dst_e)

    h1, dis = pl.pallas_call(
        _lin1_body,
        out_shape=(
            jax.ShapeDtypeStruct((NP, H), jnp.float32),
            jax.ShapeDtypeStruct((NP, H), jnp.float32),
        ),
    )(x_pad, W1, deg)

    p1 = _agg(h1, src_e, dst_e)

    h2 = pl.pallas_call(
        _lin2_body,
        out_shape=jax.ShapeDtypeStruct((NP, H), jnp.float32),
    )(p1, dis, b1r, w2p)

    p2 = _agg(h2, src_e, dst_e)

    out = pl.pallas_call(
        _fin_body,
        out_shape=jax.ShapeDtypeStruct((NP, H), jnp.float32),
    )(p2, dis, b2r)

    return out[:N, : b2.shape[0]]


# speedup vs baseline: 59.4086x; 1.0517x over previous
import functools

import jax
import jax.numpy as jnp
from jax import lax
from jax.experimental import pallas as pl
from jax.experimental.pallas import tpu as pltpu
from jax.experimental.pallas import tpu_sc as plsc

N = 10000
NP = 10240
F = 128
H = 16
NC = 2
NS = 16
LANES = 16

CH = 1296
GPT = 8
EW = CH * GPT
EP = NC * NS * EW
RPT = NP // NS
DPT = NP // (NC * NS)
DEGR = EP // (NS * CH)

_mesh = plsc.VectorSubcoreMesh(core_axis_name="c", subcore_axis_name="s")


@functools.partial(
    pl.kernel,
    out_type=jax.ShapeDtypeStruct((NP, H), jnp.float32),
    mesh=_mesh,
    scratch_types=[
        pltpu.VMEM((DEGR, CH), jnp.int32),
        pltpu.VMEM((CH,), jnp.float32),
        pltpu.VMEM((RPT,), jnp.float32),
        pltpu.VMEM((DPT,), jnp.float32),
        pltpu.VMEM((DPT, H), jnp.float32),
        pltpu.VMEM_SHARED((NP,), jnp.float32),
        pltpu.SemaphoreType.DMA,
    ],
    compiler_params=pltpu.CompilerParams(use_tc_tiling_on_sc=False),
)
def _deg_dis(dst_hbm, out_hbm, dstv, ones, zb, degb, disb, deg_sh, dsem):
    c = lax.axis_index("c")
    s = lax.axis_index("s")

    def zb_init(i, _):
        zb[pl.ds(i * LANES, LANES)] = jnp.zeros((LANES,), jnp.float32)
        return 0

    lax.fori_loop(0, RPT // LANES, zb_init, 0)

    def ones_init(i, _):
        ones[pl.ds(i * LANES, LANES)] = jnp.full((LANES,), 1.0, jnp.float32)
        return 0

    lax.fori_loop(0, CH // LANES, ones_init, 0)
    pltpu.sync_copy(zb.at[pl.ds(0, RPT)], deg_sh.at[pl.ds(s * RPT, RPT)])
    plsc.subcore_barrier()

    pltpu.sync_copy(dst_hbm.at[pl.ds(s * DEGR, DEGR)], dstv)

    descs = [pltpu.async_copy(ones, deg_sh.at[dstv.at[r]], dsem, add=True)
             for r in range(DEGR)]
    for d in descs:
        d.wait()
    plsc.subcore_barrier()

    g0 = (c * NS + s) * DPT
    pltpu.sync_copy(deg_sh.at[pl.ds(g0, DPT)], degb)

    def dis_blk(i, _):
        v = degb[pl.ds(i * LANES, LANES)]
        for k in range(LANES):
            disb[i * LANES + k] = jnp.full((LANES,), v[k], jnp.float32)
        return 0

    lax.fori_loop(0, DPT // LANES, dis_blk, 0)
    pltpu.sync_copy(disb, out_hbm.at[pl.ds(g0, DPT)])


@functools.partial(
    pl.kernel,
    out_type=jax.ShapeDtypeStruct((NC, NP, H), jnp.float32),
    mesh=_mesh,
    scratch_types=[
        pltpu.VMEM((GPT, CH), jnp.int32),
        pltpu.VMEM((GPT, CH), jnp.int32),
        [pltpu.VMEM((CH, H), jnp.float32)] * 4,
        [pltpu.SemaphoreType.DMA] * 4,
        [pltpu.SemaphoreType.DMA] * 4,
        pltpu.VMEM_SHARED((NP, H), jnp.float32),
    ],
    compiler_params=pltpu.CompilerParams(use_tc_tiling_on_sc=False),
)
def _agg(h_hbm, src_hbm, dst_hbm, out_hbm, srcv, dstv, bufs, gsems, ssems,
         acc_sh):
    c = lax.axis_index("c")
    s = lax.axis_index("s")
    rows0 = bufs[0]

    def zrow(i, _):
        rows0[i] = jnp.zeros((H,), jnp.float32)
        return 0

    lax.fori_loop(0, RPT, zrow, 0)
    pltpu.sync_copy(rows0.at[pl.ds(0, RPT)], acc_sh.at[pl.ds(s * RPT, RPT)])
    plsc.subcore_barrier()

    wid = c * NS + s
    pltpu.sync_copy(src_hbm.at[pl.ds(wid * GPT, GPT)], srcv)
    pltpu.sync_copy(dst_hbm.at[pl.ds(wid * GPT, GPT)], dstv)

    gd = [None] * GPT
    sd = [None] * GPT
    for g in range(min(2, GPT)):
        gd[g] = pltpu.async_copy(h_hbm.at[srcv.at[g]], bufs[g % 4],
                                 gsems[g % 4])
    for g in range(GPT):
        gd[g].wait()
        sd[g] = pltpu.async_copy(bufs[g % 4], acc_sh.at[dstv.at[g]],
                                 ssems[g % 4], add=True)
        ng = g + 2
        if ng < GPT:
            if ng >= 4:
                sd[ng - 4].wait()
            gd[ng] = pltpu.async_copy(h_hbm.at[srcv.at[ng]], bufs[ng % 4],
                                      gsems[ng % 4])
    for g in range(max(0, GPT - 4), GPT):
        sd[g].wait()
    plsc.subcore_barrier()
    pltpu.sync_copy(acc_sh.at[pl.ds(s * RPT, RPT)],
                    out_hbm.at[c, pl.ds(s * RPT, RPT)])


def _lin1_body(x_ref, w_ref, deg_ref, o_ref, dis_ref):
    dis = lax.rsqrt(jnp.maximum(deg_ref[...], 1.0))
    dis_ref[...] = dis
    h = jnp.dot(x_ref[...], w_ref[...], preferred_element_type=jnp.float32)
    o_ref[...] = h * dis


def _lin2_body(p_ref, dis_ref, b1_ref, w2_ref, o_ref):
    acc = p_ref[0] + p_ref[1]
    out1 = jnp.maximum(acc * dis_ref[...] + b1_ref[...], 0.0)
    h2 = jnp.dot(out1, w2_ref[...], preferred_element_type=jnp.float32)
    o_ref[...] = h2 * dis_ref[...]


def _fin_body(p_ref, dis_ref, b2_ref, o_ref):
    o_ref[...] = (p_ref[0] + p_ref[1]) * dis_ref[...] + b2_ref[...]


def kernel(x, edge_index, W1, b1, W2, b2):
    E = edge_index.shape[1]
    src = edge_index[0].astype(jnp.int32)
    dst = edge_index[1].astype(jnp.int32)
    loop_idx = jnp.arange(N, dtype=jnp.int32)
    padv = N + jnp.arange(EP - E - N, dtype=jnp.int32) % (NP - N)
    src_e = jnp.concatenate([src, loop_idx, padv]).reshape(EP // CH, CH)
    dst_e = jnp.concatenate([dst, loop_idx, padv]).reshape(EP // CH, CH)

    x_pad = jnp.pad(x, ((0, NP - N), (0, 0)))
    w2p = jnp.pad(W2, ((0, 0), (0, H - W2.shape[1])))
    b1r = b1.reshape(1, H)
    b2r = jnp.pad(b2, (0, H - b2.shape[0])).reshape(1, H)

    deg = _deg_dis(dst_e)

    h1, dis = pl.pallas_call(
        _lin1_body,
        out_shape=(
            jax.ShapeDtypeStruct((NP, H), jnp.float32),
            jax.ShapeDtypeStruct((NP, H), jnp.float32),
        ),
    )(x_pad, W1, deg)

    p1 = _agg(h1, src_e, dst_e)

    h2 = pl.pallas_call(
        _lin2_body,
        out_shape=jax.ShapeDtypeStruct((NP, H), jnp.float32),
    )(p1, dis, b1r, w2p)

    p2 = _agg(h2, src_e, dst_e)

    out = pl.pallas_call(
        _fin_body,
        out_shape=jax.ShapeDtypeStruct((NP, H), jnp.float32),
    )(p2, dis, b2r)

    return out[:N, : b2.shape[0]]
